# baseline (speedup 1.0000x reference)
import functools

import jax
import jax.numpy as jnp
from jax import lax
from jax.experimental import pallas as pl
from jax.experimental.pallas import tpu as pltpu
from jax.experimental.pallas import tpu_sc as plsc

VOCAB = 100000
D = 300
B = 4096 * 50
NC = 2
NS = 16
NW = NC * NS
BPW = B // NW
CHUNK = 64
NCHUNK = BPW // CHUNK
GRAN = 16
NG = 20
NGRAN = VOCAB * D // GRAN
G0MAX = NGRAN - NG

_MESH = plsc.VectorSubcoreMesh(core_axis_name="c", subcore_axis_name="s")


@functools.partial(
    pl.kernel,
    mesh=_MESH,
    out_type=jax.ShapeDtypeStruct((B * D,), jnp.float32),
    compiler_params=pltpu.CompilerParams(
        use_tc_tiling_on_sc=False, needs_layout_passes=False),
    scratch_types=[
        pltpu.VMEM((BPW,), jnp.int32),
        pltpu.VMEM((NG * CHUNK,), jnp.int32),
        pltpu.VMEM((NG * CHUNK, GRAN), jnp.float32),
        pltpu.VMEM((CHUNK * D,), jnp.float32),
        pltpu.VMEM((CHUNK,), jnp.int32),
        pltpu.SemaphoreType.DMA,
    ],
)
def _gather_kernel(idx_hbm, tview_hbm, out_hbm, idx_v, gidx_v, dst_v,
                   cmp_v, ph_v, sem):
    wid = lax.axis_index("s") * NC + lax.axis_index("c")
    base = wid * BPW
    pltpu.sync_copy(idx_hbm.at[wid], idx_v)

    lane = lax.iota(jnp.int32, 16)
    ngroup = CHUNK // 16

    def chunk_body(c, carry):
        for g in range(ngroup):
            t = idx_v[pl.ds(c * CHUNK + g * 16, 16)]
            q0 = t * D
            g0 = jnp.minimum(lax.shift_right_logical(q0, 4), G0MAX)
            ph_v[pl.ds(g * 16, 16)] = q0 - lax.shift_left(g0, 4)
            for j in range(NG):
                gidx_v[pl.ds(j * CHUNK + g * 16, 16)] = g0 + j
        pltpu.async_copy(tview_hbm.at[gidx_v], dst_v, sem).wait()
        qs, os_, rs = [], [], []
        for g in range(ngroup):
            qs.append(ph_v[pl.ds(g * 16, 16)])
            os_.append((g * 16 + lane) * D)
            rs.append(g * 16 + lane)

        def m_body(m, carry2):
            qo = carry2
            new = []
            for g in range(ngroup):
                q, o = qo[2 * g], qo[2 * g + 1]
                for _ in range(2):
                    j = lax.shift_right_logical(q, 4)
                    jr = lax.shift_left(j, 6) + rs[g]
                    w = lax.bitwise_and(q, 15)
                    v = plsc.load_gather(dst_v, [jr, w])
                    plsc.store_scatter(cmp_v, [o], v)
                    q = q + 1
                    o = o + 1
                new.extend((q, o))
            return tuple(new)

        init = []
        for g in range(ngroup):
            init.extend((qs[g], os_[g]))
        lax.fori_loop(0, D // 2, m_body, tuple(init))

        pltpu.sync_copy(
            cmp_v, out_hbm.at[pl.ds((base + c * CHUNK) * D, CHUNK * D)])
        return carry

    lax.fori_loop(0, NCHUNK, chunk_body, 0)


def kernel(inp, table):
    idx = inp.reshape(NW, BPW).astype(jnp.int32)
    tview = table.reshape(NGRAN, GRAN)
    out = _gather_kernel(idx, tview)
    return out.reshape(inp.shape[0], inp.shape[1], D)


# speedup vs baseline: 1.0007x; 1.0007x over previous
import functools

import jax
import jax.numpy as jnp
from jax import lax
from jax.experimental import pallas as pl
from jax.experimental.pallas import tpu as pltpu
from jax.experimental.pallas import tpu_sc as plsc

VOCAB = 100000
D = 300
B = 4096 * 50
NC = 2
NS = 16
NW = NC * NS
BPW = B // NW
CHUNK = 64
NCHUNK = BPW // CHUNK
GRAN = 16
NG = 20
NGRAN = VOCAB * D // GRAN
G0MAX = NGRAN - NG

_MESH = plsc.VectorSubcoreMesh(core_axis_name="c", subcore_axis_name="s")


@functools.partial(
    pl.kernel,
    mesh=_MESH,
    out_type=jax.ShapeDtypeStruct((B * D,), jnp.float32),
    compiler_params=pltpu.CompilerParams(
        use_tc_tiling_on_sc=False, needs_layout_passes=False,
        disable_bounds_checks=True),
    scratch_types=[
        pltpu.VMEM((BPW,), jnp.int32),
        pltpu.VMEM((NG * CHUNK,), jnp.int32),
        pltpu.VMEM((NG * CHUNK, GRAN), jnp.float32),
        pltpu.VMEM((CHUNK * D,), jnp.float32),
        pltpu.VMEM((CHUNK,), jnp.int32),
        pltpu.SemaphoreType.DMA,
    ],
)
def _gather_kernel(idx_hbm, tview_hbm, out_hbm, idx_v, gidx_v, dst_v,
                   cmp_v, ph_v, sem):
    wid = lax.axis_index("s") * NC + lax.axis_index("c")
    base = wid * BPW
    pltpu.sync_copy(idx_hbm.at[wid], idx_v)

    lane = lax.iota(jnp.int32, 16)
    ngroup = CHUNK // 16

    def chunk_body(c, carry):
        for g in range(ngroup):
            t = idx_v[pl.ds(c * CHUNK + g * 16, 16)]
            q0 = t * D
            g0 = jnp.minimum(lax.shift_right_logical(q0, 4), G0MAX)
            ph_v[pl.ds(g * 16, 16)] = q0 - lax.shift_left(g0, 4)
            for j in range(NG):
                gidx_v[pl.ds(j * CHUNK + g * 16, 16)] = g0 + j
        pltpu.async_copy(tview_hbm.at[gidx_v], dst_v, sem).wait()
        qs, os_, rs = [], [], []
        for g in range(ngroup):
            qs.append(ph_v[pl.ds(g * 16, 16)])
            os_.append((g * 16 + lane) * D)
            rs.append(g * 16 + lane)

        def m_body(m, carry2):
            qo = carry2
            new = []
            for g in range(ngroup):
                q, o = qo[2 * g], qo[2 * g + 1]
                for _ in range(2):
                    j = lax.shift_right_logical(q, 4)
                    jr = lax.shift_left(j, 6) + rs[g]
                    w = lax.bitwise_and(q, 15)
                    v = plsc.load_gather(dst_v, [jr, w])
                    plsc.store_scatter(cmp_v, [o], v)
                    q = q + 1
                    o = o + 1
                new.extend((q, o))
            return tuple(new)

        init = []
        for g in range(ngroup):
            init.extend((qs[g], os_[g]))
        lax.fori_loop(0, D // 2, m_body, tuple(init))

        pltpu.sync_copy(
            cmp_v, out_hbm.at[pl.ds((base + c * CHUNK) * D, CHUNK * D)])
        return carry

    lax.fori_loop(0, NCHUNK, chunk_body, 0)


def kernel(inp, table):
    idx = inp.reshape(NW, BPW).astype(jnp.int32)
    tview = table.reshape(NGRAN, GRAN)
    out = _gather_kernel(idx, tview)
    return out.reshape(inp.shape[0], inp.shape[1], D)


# speedup vs baseline: 1.0890x; 1.0882x over previous
import functools

import jax
import jax.numpy as jnp
from jax import lax
from jax.experimental import pallas as pl
from jax.experimental.pallas import tpu as pltpu
from jax.experimental.pallas import tpu_sc as plsc

VOCAB = 100000
D = 300
B = 4096 * 50
NC = 2
NS = 16
NW = NC * NS
BPW = B // NW
CHUNK = 64
NCHUNK = BPW // CHUNK
GRAN = 320
NGRAN = VOCAB * D // GRAN
G0MAX = NGRAN - 2

_MESH = plsc.VectorSubcoreMesh(core_axis_name="c", subcore_axis_name="s")


@functools.partial(
    pl.kernel,
    mesh=_MESH,
    out_type=jax.ShapeDtypeStruct((B * D,), jnp.float32),
    compiler_params=pltpu.CompilerParams(
        use_tc_tiling_on_sc=False, needs_layout_passes=False),
    scratch_types=[
        pltpu.VMEM((BPW,), jnp.int32),
        pltpu.VMEM((2 * CHUNK,), jnp.int32),
        pltpu.VMEM((2 * CHUNK,), jnp.int32),
        pltpu.VMEM((2 * CHUNK, GRAN), jnp.float32),
        pltpu.VMEM((2 * CHUNK, GRAN), jnp.float32),
        pltpu.VMEM((CHUNK,), jnp.int32),
        pltpu.VMEM((CHUNK,), jnp.int32),
        pltpu.VMEM((CHUNK * D + 16,), jnp.float32),
        pltpu.SemaphoreType.DMA,
        pltpu.SemaphoreType.DMA,
    ],
)
def _gather_kernel(idx_hbm, tview_hbm, out_hbm, idx_v, gidx0, gidx1,
                   buf0, buf1, ph0, ph1, cmp_v, sem0, sem1):
    wid = lax.axis_index("s") * NC + lax.axis_index("c")
    base = wid * BPW
    pltpu.sync_copy(idx_hbm.at[wid], idx_v)

    lane = lax.iota(jnp.int32, 16)
    ngroup = CHUNK // 16
    gidxs = (gidx0, gidx1)
    bufs = (buf0, buf1)
    phs = (ph0, ph1)
    sems = (sem0, sem1)

    def build(c, gidx_v, ph_v):
        for g in range(ngroup):
            t = idx_v[pl.ds(c * CHUNK + g * 16, 16)]
            q = t * D
            u = lax.shift_right_logical(q, 6)
            g0 = jnp.minimum(
                ((u.astype(jnp.float32) + 0.5) * 0.2).astype(jnp.int32),
                G0MAX)
            ph_v[pl.ds(g * 16, 16)] = q - g0 * GRAN
            pos = (g * 16 + lane) * 2
            plsc.store_scatter(gidx_v, [pos], g0)
            plsc.store_scatter(gidx_v, [pos + 1], g0 + 1)

    def fire(c, par):
        pltpu.async_copy(tview_hbm.at[gidxs[par]], bufs[par], sems[par])

    def wait(par):
        pltpu.make_async_copy(
            tview_hbm.at[gidxs[par]], bufs[par], sems[par]).wait()

    def realign(c, buf_v, ph_v):
        def row_body(r, carry):
            grp16 = lax.shift_right_logical(r, 4) * 16
            l = lax.bitwise_and(r, 15)
            pv = ph_v[pl.ds(grp16, 16)]
            p = lax.reduce_max(jnp.where(lane == l, pv, 0), (0,))
            qbase = p + lane
            slot0 = r * 2
            dst0 = r * D
            for k in range(D // 16 + 1):
                q = qbase + 16 * k
                ge = q >= GRAN
                slot = slot0 + ge.astype(jnp.int32)
                w = jnp.where(ge, q - GRAN, q)
                cmp_v[pl.ds(dst0 + 16 * k, 16)] = plsc.load_gather(
                    buf_v, [slot, w])
            return carry

        lax.fori_loop(0, CHUNK, row_body, 0)
        pltpu.sync_copy(
            cmp_v.at[pl.ds(0, CHUNK * D)],
            out_hbm.at[pl.ds((base + c * CHUNK) * D, CHUNK * D)])

    build(0, gidx0, ph0)
    fire(0, 0)

    def chunk_body(c, carry):
        for par in range(2):
            @pl.when(lax.rem(c, 2) == par)
            def _():
                @pl.when(c + 1 < NCHUNK)
                def _():
                    build(c + 1, gidxs[1 - par], phs[1 - par])
                    fire(c + 1, 1 - par)
                wait(par)
                realign(c, bufs[par], phs[par])
        return carry

    lax.fori_loop(0, NCHUNK, chunk_body, 0)


def kernel(inp, table):
    idx = inp.reshape(NW, BPW).astype(jnp.int32)
    tview = table.reshape(NGRAN, GRAN)
    out = _gather_kernel(idx, tview)
    return out.reshape(inp.shape[0], inp.shape[1], D)


# speedup vs baseline: 1.1081x; 1.0176x over previous
import functools

import jax
import jax.numpy as jnp
from jax import lax
from jax.experimental import pallas as pl
from jax.experimental.pallas import tpu as pltpu
from jax.experimental.pallas import tpu_sc as plsc

VOCAB = 100000
D = 300
B = 4096 * 50
NC = 2
NS = 16
NW = NC * NS
BPW = B // NW
CHUNK = 64
NCHUNK = BPW // CHUNK
GRAN = 320
NGRAN = VOCAB * D // GRAN
G0MAX = NGRAN - 2

_MESH = plsc.VectorSubcoreMesh(core_axis_name="c", subcore_axis_name="s")


@functools.partial(
    pl.kernel,
    mesh=_MESH,
    out_type=jax.ShapeDtypeStruct((B * D,), jnp.float32),
    compiler_params=pltpu.CompilerParams(
        use_tc_tiling_on_sc=False, needs_layout_passes=False),
    scratch_types=[
        pltpu.VMEM((2 * BPW,), jnp.int32),
        pltpu.VMEM((BPW,), jnp.int32),
        pltpu.VMEM((2 * CHUNK, GRAN), jnp.float32),
        pltpu.VMEM((2 * CHUNK, GRAN), jnp.float32),
        pltpu.VMEM((CHUNK * D + 16,), jnp.float32),
        pltpu.SemaphoreType.DMA,
        pltpu.SemaphoreType.DMA,
    ],
)
def _gather_kernel(gidx_hbm, ph_hbm, tview_hbm, out_hbm, gidx_v, ph_v,
                   buf0, buf1, cmp_v, sem0, sem1):
    wid = lax.axis_index("s") * NC + lax.axis_index("c")
    base = wid * BPW
    pltpu.sync_copy(gidx_hbm.at[wid], gidx_v)
    pltpu.sync_copy(ph_hbm.at[wid], ph_v)

    lane = lax.iota(jnp.int32, 16)
    bufs = (buf0, buf1)
    sems = (sem0, sem1)

    def fire(c, par):
        pltpu.async_copy(
            tview_hbm.at[gidx_v.at[pl.ds(c * 2 * CHUNK, 2 * CHUNK)]],
            bufs[par], sems[par])

    def wait(par):
        pltpu.make_async_copy(
            tview_hbm.at[gidx_v.at[pl.ds(0, 2 * CHUNK)]],
            bufs[par], sems[par]).wait()

    def realign(c, buf_v):
        def row_body(r, carry):
            grp16 = lax.shift_right_logical(r, 4) * 16
            l = lax.bitwise_and(r, 15)
            pv = ph_v[pl.ds(c * CHUNK + grp16, 16)]
            p = lax.reduce_max(jnp.where(lane == l, pv, 0), (0,))
            qbase = p + lane
            slot0 = r * 2
            dst0 = r * D
            for k in range(D // 16 + 1):
                q = qbase + 16 * k
                ge = q >= GRAN
                slot = slot0 + ge.astype(jnp.int32)
                w = jnp.where(ge, q - GRAN, q)
                cmp_v[pl.ds(dst0 + 16 * k, 16)] = plsc.load_gather(
                    buf_v, [slot, w])
            return carry

        lax.fori_loop(0, CHUNK, row_body, 0)
        pltpu.sync_copy(
            cmp_v.at[pl.ds(0, CHUNK * D)],
            out_hbm.at[pl.ds((base + c * CHUNK) * D, CHUNK * D)])

    fire(0, 0)

    def chunk_body(c, carry):
        for par in range(2):
            @pl.when(lax.rem(c, 2) == par)
            def _():
                @pl.when(c + 1 < NCHUNK)
                def _():
                    fire(c + 1, 1 - par)
                wait(par)
                realign(c, bufs[par])
        return carry

    lax.fori_loop(0, NCHUNK, chunk_body, 0)


def kernel(inp, table):
    idx = inp.reshape(-1).astype(jnp.int32)
    q = idx * D
    g0 = jnp.minimum(q // GRAN, G0MAX)
    ph = (q - g0 * GRAN).reshape(NW, BPW)
    gidx = jnp.stack([g0, g0 + 1], axis=-1).reshape(NW, 2 * BPW)
    tview = table.reshape(NGRAN, GRAN)
    out = _gather_kernel(gidx, ph, tview)
    return out.reshape(inp.shape[0], inp.shape[1], D)
